# baseline (speedup 1.0000x reference)
import functools

import jax
import jax.numpy as jnp
from jax import lax
from jax.experimental import pallas as pl
from jax.experimental.pallas import tpu as pltpu
from jax.experimental.pallas import tpu_sc as plsc

B = 16384
L = 20
EMB = 32
NC = 2
NS = 16
NW = NC * NS
BPW = B // NW

_mesh = plsc.VectorSubcoreMesh(
    core_axis_name="c", subcore_axis_name="s", num_cores=NC, num_subcores=NS
)

_f32 = jnp.float32


@functools.partial(
    pl.kernel,
    out_type=[
        jax.ShapeDtypeStruct((B, EMB), _f32),
        jax.ShapeDtypeStruct((B, EMB), _f32),
        jax.ShapeDtypeStruct((B, EMB), _f32),
    ],
    mesh=_mesh,
    compiler_params=pltpu.CompilerParams(
        use_tc_tiling_on_sc=False, needs_layout_passes=False
    ),
    scratch_types=[
        pltpu.VMEM((BPW * L,), jnp.int32),
        pltpu.VMEM((L, BPW), jnp.int32),
        pltpu.VMEM((BPW,), jnp.int32),
        pltpu.VMEM((BPW,), jnp.int32),
        pltpu.VMEM((BPW, EMB), _f32),
        pltpu.VMEM((BPW, EMB), _f32),
        pltpu.VMEM((BPW, EMB), _f32),
        pltpu.VMEM((BPW,), _f32),
        pltpu.VMEM((BPW,), _f32),
        pltpu.VMEM((EMB,), _f32),
        pltpu.SemaphoreType.DMA,
        pltpu.SemaphoreType.DMA,
    ],
)
def _sc_embed(
    gid_hbm, tid_hbm, aid_hbm, gtab, ttab, atab, gout, tout, aout,
    gid_raw, gid_t, tid_v, aid_v, acc_v, t_v, a_v, n0_v, rec_v, row0_v,
    gsem, tsem,
):
    wid = lax.axis_index("c") * NS + lax.axis_index("s")
    base = wid * BPW

    pltpu.sync_copy(tid_hbm.at[pl.ds(base, BPW)], tid_v)
    pltpu.sync_copy(aid_hbm.at[pl.ds(base, BPW)], aid_v)
    pltpu.async_copy(ttab.at[tid_v], t_v, tsem)
    pltpu.async_copy(atab.at[aid_v], a_v, tsem)
    pltpu.sync_copy(gid_hbm.at[pl.ds(base * L, BPW * L)], gid_raw)
    pltpu.sync_copy(gtab.at[0], row0_v)

    lanesL = lax.broadcasted_iota(jnp.int32, (16,), 0) * L

    def transpose_col(l):
        for v in range(BPW // 16):
            gid_t[l, pl.ds(v * 16, 16)] = plsc.load_gather(
                gid_raw, [lanesL + (v * 16 * L + l)]
            )

    transpose_col(0)
    pltpu.async_copy(gtab.at[gid_t.at[0]], acc_v, gsem)
    for l in range(1, L):
        transpose_col(l)
    pltpu.make_async_copy(gtab.at[gid_t.at[0]], acc_v, gsem).wait()

    def fire(l, _):
        pltpu.async_copy(gtab.at[gid_t.at[l]], acc_v, gsem, add=True)
        return 0

    lax.fori_loop(1, L, fire, 0)

    def count_body(i, _):
        off = i * 16
        acc = jnp.zeros((16,), _f32)
        for l in range(L):
            ids = gid_t[l, pl.ds(off, 16)]
            acc = acc + jnp.where(ids == 0, 1.0, 0.0).astype(_f32)
        n0_v[pl.ds(off, 16)] = acc
        cnt = jnp.float32(L) - acc
        rec_v[pl.ds(off, 16)] = jnp.where(
            cnt > 0.5, jnp.float32(1.0) / cnt, jnp.float32(0.0)
        )
        return 0

    lax.fori_loop(0, BPW // 16, count_body, 0)

    pltpu.make_async_copy(ttab.at[tid_v], t_v, tsem).wait()
    pltpu.make_async_copy(atab.at[aid_v], a_v, tsem).wait()
    pltpu.sync_copy(t_v, tout.at[pl.ds(base, BPW), :])
    pltpu.sync_copy(a_v, aout.at[pl.ds(base, BPW), :])

    def drain(i, _):
        pltpu.make_async_copy(gtab.at[gid_t.at[0]], acc_v, gsem).wait()
        return 0

    lax.fori_loop(1, L, drain, 0)

    r0a = row0_v[pl.ds(0, 16)]
    r0b = row0_v[pl.ds(16, 16)]

    def norm(g, _):
        n0g = n0_v[pl.ds(g * 16, 16)]
        recg = rec_v[pl.ds(g * 16, 16)]
        for j in range(16):
            r = g * 16 + j
            n0 = n0g[j]
            rec = recg[j]
            v0 = acc_v[r, pl.ds(0, 16)]
            v1 = acc_v[r, pl.ds(16, 16)]
            acc_v[r, pl.ds(0, 16)] = (v0 - n0 * r0a) * rec
            acc_v[r, pl.ds(16, 16)] = (v1 - n0 * r0b) * rec
        return 0

    lax.fori_loop(0, BPW // 16, norm, 0)

    pltpu.sync_copy(acc_v, gout.at[pl.ds(base, BPW), :])


def kernel(genre_ids, type_ids, audience_ids, genre_table, type_table,
           audience_table):
    gids = genre_ids.astype(jnp.int32).reshape(-1)
    tids = type_ids.astype(jnp.int32).reshape(-1)
    aids = audience_ids.astype(jnp.int32).reshape(-1)
    g, t, a = _sc_embed(gids, tids, aids, genre_table, type_table,
                        audience_table)
    return jnp.concatenate([g, t, a], axis=1)


# speedup vs baseline: 1.8034x; 1.8034x over previous
import functools

import jax
import jax.numpy as jnp
from jax import lax
from jax.experimental import pallas as pl
from jax.experimental.pallas import tpu as pltpu
from jax.experimental.pallas import tpu_sc as plsc

B = 16384
L = 20
EMB = 32
TYPE_V = 101
AUD_V = 21
NC = 2
NS = 16
NW = NC * NS
BPW = B // NW

_mesh = plsc.VectorSubcoreMesh(
    core_axis_name="c", subcore_axis_name="s", num_cores=NC, num_subcores=NS
)

_f32 = jnp.float32


@functools.partial(
    pl.kernel,
    out_type=jax.ShapeDtypeStruct((B, 3 * EMB), _f32),
    mesh=_mesh,
    compiler_params=pltpu.CompilerParams(use_tc_tiling_on_sc=False),
    scratch_types=[
        pltpu.VMEM((L, 1, BPW), jnp.int32),
        pltpu.VMEM((BPW,), jnp.int32),
        pltpu.VMEM((BPW,), jnp.int32),
        pltpu.VMEM((BPW, EMB), _f32),
        pltpu.VMEM((TYPE_V, EMB), _f32),
        pltpu.VMEM((AUD_V, EMB), _f32),
        pltpu.VMEM((BPW,), _f32),
        pltpu.VMEM((BPW,), _f32),
        pltpu.VMEM((EMB,), _f32),
        pltpu.VMEM((BPW, 3 * EMB), _f32),
        pltpu.SemaphoreType.DMA,
    ],
)
def _sc_embed(
    gidx_hbm, tid_hbm, aid_hbm, gtab, ttab, atab, out_hbm,
    gid_v, tid_v, aid_v, acc_v, ttab_v, atab_v, n0_v, rec_v, row0_v,
    out_v, gsem,
):
    wid = lax.axis_index("c") * NS + lax.axis_index("s")
    base = wid * BPW

    pltpu.sync_copy(gidx_hbm.at[:, pl.ds(wid, 1), :], gid_v)

    pltpu.async_copy(gtab.at[gid_v.at[0, 0]], acc_v, gsem)

    pltpu.sync_copy(tid_hbm.at[pl.ds(base, BPW)], tid_v)
    pltpu.sync_copy(aid_hbm.at[pl.ds(base, BPW)], aid_v)
    pltpu.sync_copy(ttab, ttab_v)
    pltpu.sync_copy(atab, atab_v)
    pltpu.sync_copy(gtab.at[0], row0_v)

    pltpu.make_async_copy(gtab.at[gid_v.at[0, 0]], acc_v, gsem).wait()

    def fire(l, _):
        pltpu.async_copy(gtab.at[gid_v.at[l, 0]], acc_v, gsem, add=True)
        return 0

    lax.fori_loop(1, L, fire, 0)

    def count_body(g, _):
        off = g * 16
        acc = jnp.zeros((16,), _f32)
        for l in range(L):
            ids = gid_v[l, 0, pl.ds(off, 16)]
            acc = acc + jnp.where(ids == 0, 1.0, 0.0).astype(_f32)
        n0_v[pl.ds(off, 16)] = acc
        cnt = jnp.float32(L) - acc
        rec_v[pl.ds(off, 16)] = jnp.where(
            cnt > 0.5, jnp.float32(1.0) / cnt, jnp.float32(0.0)
        )
        tidg = tid_v[pl.ds(off, 16)]
        aidg = aid_v[pl.ds(off, 16)]
        for j in range(16):
            r = off + j
            t = tidg[j]
            a = aidg[j]
            out_v[r, pl.ds(EMB, 16)] = ttab_v[t, pl.ds(0, 16)]
            out_v[r, pl.ds(EMB + 16, 16)] = ttab_v[t, pl.ds(16, 16)]
            out_v[r, pl.ds(2 * EMB, 16)] = atab_v[a, pl.ds(0, 16)]
            out_v[r, pl.ds(2 * EMB + 16, 16)] = atab_v[a, pl.ds(16, 16)]
        return 0

    lax.fori_loop(0, BPW // 16, count_body, 0)

    def drain(l, _):
        pltpu.make_async_copy(gtab.at[gid_v.at[0, 0]], acc_v, gsem).wait()
        return 0

    lax.fori_loop(1, L, drain, 0)

    r0a = row0_v[pl.ds(0, 16)]
    r0b = row0_v[pl.ds(16, 16)]

    def norm(g, _):
        off = g * 16
        n0g = n0_v[pl.ds(off, 16)]
        recg = rec_v[pl.ds(off, 16)]
        for j in range(16):
            r = off + j
            n0 = n0g[j]
            rec = recg[j]
            v0 = acc_v[r, pl.ds(0, 16)]
            v1 = acc_v[r, pl.ds(16, 16)]
            out_v[r, pl.ds(0, 16)] = (v0 - n0 * r0a) * rec
            out_v[r, pl.ds(16, 16)] = (v1 - n0 * r0b) * rec
        return 0

    lax.fori_loop(0, BPW // 16, norm, 0)

    pltpu.sync_copy(out_v, out_hbm.at[pl.ds(base, BPW), :])


def kernel(genre_ids, type_ids, audience_ids, genre_table, type_table,
           audience_table):
    gids = genre_ids.astype(jnp.int32)
    tids = type_ids.astype(jnp.int32)
    aids = audience_ids.astype(jnp.int32)
    gidx = gids.T.reshape(L, NW, BPW)
    return _sc_embed(gidx, tids, aids, genre_table, type_table,
                     audience_table)
